# baseline (speedup 1.0000x reference)
import functools

import jax
import jax.numpy as jnp
from jax import lax
from jax.experimental import pallas as pl
from jax.experimental.pallas import tpu as pltpu
from jax.experimental.pallas import tpu_sc as plsc


def _sc_gather(hidden, ent_emb, idx_vi3, idx_vj3, idx_ent3, e_per_w, ent_w,
               do_ent):
    NW, n_ch, CH = idx_vi3.shape
    E = NW * e_per_w
    N, D = hidden.shape
    NE = ent_emb.shape[0]
    n_full = e_per_w // CH
    tail = e_per_w - n_full * CH
    assert n_ch == n_full + (1 if tail else 0)
    ENT_W, ent_ch, _ = idx_ent3.shape
    if not do_ent:
        ENT_W = 0

    mesh = plsc.VectorSubcoreMesh(core_axis_name="c", subcore_axis_name="s")

    @functools.partial(
        pl.kernel,
        out_type=(
            jax.ShapeDtypeStruct((E, D), jnp.float32),
            jax.ShapeDtypeStruct((E, D), jnp.float32),
            jax.ShapeDtypeStruct((NE, D), jnp.float32),
        ),
        mesh=mesh,
        scratch_types=[
            pltpu.VMEM((n_ch, CH), jnp.int32),
            pltpu.VMEM((n_ch, CH), jnp.int32),
            pltpu.VMEM((3, CH, D), jnp.float32),
            pltpu.VMEM((3, CH, D), jnp.float32),
            pltpu.SemaphoreType.DMA((3,)),
            pltpu.SemaphoreType.DMA((3,)),
        ],
    )
    def gather_kernel(hid_h, ent_h, ivi_h, ivj_h, ient_h,
                      ovi_h, ovj_h, oent_h,
                      ivi_v, ivj_v, rvi_v, rvj_v, g_sem, w_sem):
        wid = lax.axis_index("s") * 2 + lax.axis_index("c")
        base = wid * e_per_w

        pltpu.sync_copy(ivi_h.at[wid], ivi_v)
        pltpu.sync_copy(ivj_h.at[wid], ivj_v)

        def start_gather(i, slot):
            pltpu.async_copy(hid_h.at[ivi_v.at[i]], rvi_v.at[slot],
                             g_sem.at[slot])
            pltpu.async_copy(hid_h.at[ivj_v.at[i]], rvj_v.at[slot],
                             g_sem.at[slot])

        def wait_gather(i, slot):
            pltpu.make_async_copy(hid_h.at[ivi_v.at[i]], rvi_v.at[slot],
                                  g_sem.at[slot]).wait()
            pltpu.make_async_copy(hid_h.at[ivj_v.at[i]], rvj_v.at[slot],
                                  g_sem.at[slot]).wait()

        def start_write(i, slot, w=CH):
            off = base + i * CH
            pltpu.async_copy(rvi_v.at[slot, pl.ds(0, w)],
                             ovi_h.at[pl.ds(off, w)], w_sem.at[slot])
            pltpu.async_copy(rvj_v.at[slot, pl.ds(0, w)],
                             ovj_h.at[pl.ds(off, w)], w_sem.at[slot])

        def wait_write(i, slot, w=CH):
            off = base + i * CH
            pltpu.make_async_copy(rvi_v.at[slot, pl.ds(0, w)],
                                  ovi_h.at[pl.ds(off, w)],
                                  w_sem.at[slot]).wait()
            pltpu.make_async_copy(rvj_v.at[slot, pl.ds(0, w)],
                                  ovj_h.at[pl.ds(off, w)],
                                  w_sem.at[slot]).wait()

        start_gather(0, 0)
        start_gather(1, 1)

        def ebody(i, _):
            slot = lax.rem(i, 3)
            wait_gather(i, slot)
            start_write(i, slot)

            @pl.when(i + 2 < n_ch)
            def _():
                fslot = lax.rem(i + 2, 3)

                @pl.when(i >= 1)
                def _():
                    wait_write(i - 1, fslot)
                start_gather(i + 2, fslot)
            return ()

        lax.fori_loop(0, n_full, ebody, (), unroll=False)
        if tail:
            tslot = n_full % 3
            wait_gather(n_full, tslot)
            start_write(n_full, tslot, tail)
            wait_write(n_full, tslot, tail)
        else:
            wait_write(n_full - 3, (n_full - 3) % 3)
        wait_write(n_full - 2, (n_full - 2) % 3)
        wait_write(n_full - 1, (n_full - 1) % 3)

        if ENT_W:
            @pl.when(wid < ENT_W)
            def _():
                pltpu.sync_copy(ient_h.at[wid], ivi_v.at[pl.ds(0, ent_ch)])

                ech = ent_w // ent_ch

                def nbody(i, _):
                    off = wid * ent_w + i * ech
                    pltpu.async_copy(ent_h.at[ivi_v.at[i]], rvi_v.at[0],
                                     g_sem.at[0]).wait()
                    pltpu.sync_copy(rvi_v.at[0, pl.ds(0, ech)],
                                    oent_h.at[pl.ds(off, ech)])
                    return ()
                lax.fori_loop(0, ent_ch, nbody, (), unroll=False)

    return gather_kernel(hidden, ent_emb, idx_vi3, idx_vj3, idx_ent3)


def _tc_edge(hvi, hvj, rel2d, tatt2d, rel_emb, f_msg_ws, f_msg_b,
             g_msg_W, g_msg_b, f_att_ws):
    E, D = hvi.shape
    R = rel_emb.shape[0]
    BLK = 8000
    n_blk = E // BLK
    assert n_blk * BLK == E

    def body(hvi_r, hvj_r, rel_r, tatt_r, remb_r, fw_r, fb_r, gw_r, gb_r,
             aw_r, out_r):
        rel = rel_r[...]
        onehot = (rel == lax.broadcasted_iota(jnp.int32, (BLK, R), 1)
                  ).astype(jnp.float32)
        rel_v = jnp.dot(onehot, remb_r[...],
                        preferred_element_type=jnp.float32)
        hvi_b = hvi_r[...]
        hvj_b = hvj_r[...]
        w0 = fw_r[0]
        w1 = fw_r[1]
        w2 = fw_r[2]
        msg = jnp.tanh(hvj_b * (w0 + rel_v * (w1 + hvi_b * w2)) + fb_r[...])
        msg = msg + jnp.tanh(
            jnp.dot(msg, gw_r[...], preferred_element_type=jnp.float32)
            + gb_r[...])
        out_r[...] = msg * tatt_r[...] * aw_r[...]

    return pl.pallas_call(
        body,
        grid=(n_blk,),
        in_specs=[
            pl.BlockSpec((BLK, D), lambda i: (i, 0)),
            pl.BlockSpec((BLK, D), lambda i: (i, 0)),
            pl.BlockSpec((BLK, 1), lambda i: (i, 0)),
            pl.BlockSpec((BLK, 1), lambda i: (i, 0)),
            pl.BlockSpec((R, D), lambda i: (0, 0)),
            pl.BlockSpec((3, D), lambda i: (0, 0)),
            pl.BlockSpec((D,), lambda i: (0,)),
            pl.BlockSpec((D, D), lambda i: (0, 0)),
            pl.BlockSpec((D,), lambda i: (0,)),
            pl.BlockSpec((1, D), lambda i: (0, 0)),
        ],
        out_specs=pl.BlockSpec((BLK, D), lambda i: (i, 0)),
        out_shape=jax.ShapeDtypeStruct((E, D), jnp.float32),
    )(hvi, hvj, rel2d, tatt2d, rel_emb, f_msg_ws, f_msg_b, g_msg_W,
      g_msg_b, f_att_ws)


def _sc_scatter(att_msg, idx_dst3, zeros_nd, e_per_w):
    E, D = att_msg.shape
    N = zeros_nd.shape[0]
    NW, n_ch, CH = idx_dst3.shape
    NS = 16
    n_full = e_per_w // CH
    tail = e_per_w - n_full * CH
    assert n_ch == n_full + (1 if tail else 0) and e_per_w * NW == E
    rows_per_s = (N // NS) // 8 * 8
    rows_tail = N - NS * rows_per_s

    mesh = plsc.VectorSubcoreMesh(core_axis_name="c", subcore_axis_name="s")

    @functools.partial(
        pl.kernel,
        out_type=jax.ShapeDtypeStruct((2 * N, D), jnp.float32),
        mesh=mesh,
        scratch_types=[
            pltpu.VMEM_SHARED((N, D), jnp.float32),
            pltpu.VMEM((n_ch, CH), jnp.int32),
            pltpu.VMEM((2, CH, D), jnp.float32),
            pltpu.SemaphoreType.DMA((2,)),
        ],
    )
    def scatter_kernel(msg_h, idx_h, zero_h, out_h, acc_sh, idx_v, rows_v,
                       l_sem):
        cid = lax.axis_index("c")
        sid = lax.axis_index("s")
        wid = sid * 2 + cid
        base = wid * e_per_w

        pltpu.sync_copy(idx_h.at[wid], idx_v)

        def start_load(i, slot, w=CH):
            off = base + i * CH
            pltpu.async_copy(msg_h.at[pl.ds(off, w)],
                             rows_v.at[slot, pl.ds(0, w)], l_sem.at[slot])

        def wait_load(i, slot, w=CH):
            off = base + i * CH
            pltpu.make_async_copy(msg_h.at[pl.ds(off, w)],
                                  rows_v.at[slot, pl.ds(0, w)],
                                  l_sem.at[slot]).wait()

        @pl.when(sid == 0)
        def _():
            pltpu.sync_copy(zero_h, acc_sh)
        start_load(0, 0)
        plsc.subcore_barrier()

        def ebody(i, _):
            slot = lax.rem(i, 2)

            @pl.when(i + 1 < n_full)
            def _():
                start_load(i + 1, lax.rem(i + 1, 2))
            wait_load(i, slot)
            pltpu.sync_copy(rows_v.at[slot], acc_sh.at[idx_v.at[i]],
                            add=True)
            return ()

        lax.fori_loop(0, n_full, ebody, (), unroll=False)
        if tail:
            tslot = n_full % 2
            start_load(n_full, tslot, tail)
            wait_load(n_full, tslot, tail)
            pltpu.sync_copy(rows_v.at[tslot, pl.ds(0, tail)],
                            acc_sh.at[idx_v.at[n_full, pl.ds(0, tail)]],
                            add=True)
        plsc.subcore_barrier()

        out_base = cid * N + sid * rows_per_s
        pltpu.sync_copy(acc_sh.at[pl.ds(sid * rows_per_s, rows_per_s)],
                        out_h.at[pl.ds(out_base, rows_per_s)])

        @pl.when(sid == 0)
        def _():
            pltpu.sync_copy(
                acc_sh.at[pl.ds(NS * rows_per_s, rows_tail)],
                out_h.at[pl.ds(cid * N + NS * rows_per_s, rows_tail)])

    return scatter_kernel(att_msg, idx_dst3, zeros_nd)


def _tc_node(partials0, partials1, hidden, ent_g, hidden_uncon, natt2d,
             proj_W, proj_b, f_hid_ws, f_hid_b, g_hid_W, g_hid_b, gru_W,
             gru_b):
    N, D = hidden.shape
    BLK = 1000
    n_blk = N // BLK
    assert n_blk * BLK == N

    def body(p0_r, p1_r, p2_r, p3_r, hid_r, ent_r, hu_r, natt_r, pw_r, pb_r,
             fw_r, fb_r, gw_r, gb_r, uw_r, ub_r, out_r):
        u = (p0_r[...] + p1_r[...]) + (p2_r[...] + p3_r[...])
        h = hid_r[...]
        ent = ent_r[...]
        uc = jnp.tanh(
            jnp.dot(hu_r[...], pw_r[...], preferred_element_type=jnp.float32)
            + pb_r[...]) * natt_r[...]
        cand = jnp.tanh(
            u * (fw_r[0] + ent * fw_r[1] + uc * fw_r[2])
            + h * (fw_r[3] + ent * fw_r[4] + uc * fw_r[5])
            + ent * fw_r[6] + uc * fw_r[7] + ent * uc * fw_r[8]
            + fb_r[...])
        cand = cand + jnp.tanh(
            jnp.dot(cand, gw_r[...], preferred_element_type=jnp.float32)
            + gb_r[...])
        z = jax.nn.sigmoid(
            jnp.dot(h, uw_r[0], preferred_element_type=jnp.float32)
            + jnp.dot(cand, uw_r[1], preferred_element_type=jnp.float32)
            + ub_r[...])
        out_r[...] = (1.0 - z) * h + z * cand

    gru_W3 = gru_W.reshape(2, D, D)

    return pl.pallas_call(
        body,
        grid=(n_blk,),
        in_specs=[
            pl.BlockSpec((BLK, D), lambda i: (i, 0)),
            pl.BlockSpec((BLK, D), lambda i: (i + n_blk, 0)),
            pl.BlockSpec((BLK, D), lambda i: (i, 0)),
            pl.BlockSpec((BLK, D), lambda i: (i + n_blk, 0)),
            pl.BlockSpec((BLK, D), lambda i: (i, 0)),
            pl.BlockSpec((BLK, D), lambda i: (i, 0)),
            pl.BlockSpec((BLK, D), lambda i: (i, 0)),
            pl.BlockSpec((BLK, 1), lambda i: (i, 0)),
            pl.BlockSpec((D, D), lambda i: (0, 0)),
            pl.BlockSpec((D,), lambda i: (0,)),
            pl.BlockSpec((9, D), lambda i: (0, 0)),
            pl.BlockSpec((D,), lambda i: (0,)),
            pl.BlockSpec((D, D), lambda i: (0, 0)),
            pl.BlockSpec((D,), lambda i: (0,)),
            pl.BlockSpec((2, D, D), lambda i: (0, 0, 0)),
            pl.BlockSpec((D,), lambda i: (0,)),
        ],
        out_specs=pl.BlockSpec((BLK, D), lambda i: (i, 0)),
        out_shape=jax.ShapeDtypeStruct((N, D), jnp.float32),
    )(partials0, partials0, partials1, partials1, hidden, ent_g,
      hidden_uncon, natt2d, proj_W, proj_b, f_hid_ws, f_hid_b, g_hid_W,
      g_hid_b, gru_W3, gru_b)


def kernel(hidden, seen_edges, trans_attention, node_attention, hidden_uncon,
           memorized_nodes, ent_emb, rel_emb, f_msg_ws, f_msg_b, g_msg_W,
           g_msg_b, f_att_ws, f_hid_ws, f_hid_b, g_hid_W, g_hid_b, proj_W,
           proj_b, gru_W, gru_b):
    E = seen_edges.shape[0]
    N, D = hidden.shape

    NW, CH, K = 32, 80, 2
    EH = E // K
    e_per_w = EH // NW
    n_ch = -(-e_per_w // CH)

    def _pad3(idx):
        a = idx.reshape(NW, e_per_w)
        a = jnp.pad(a, ((0, 0), (0, n_ch * CH - e_per_w)))
        return a.reshape(NW, n_ch, CH)

    idx_vi = seen_edges[:, 6].astype(jnp.int32)
    idx_vj = seen_edges[:, 7].astype(jnp.int32)
    rel2d = seen_edges[:, 3].astype(jnp.int32).reshape(E, 1)
    ENT_W, ent_w = 25, N // 25
    idx_ent3 = memorized_nodes[:, 1].astype(jnp.int32).reshape(ENT_W, -1, CH)
    tatt2d = trans_attention.reshape(E, 1)
    natt2d = node_attention.reshape(N, 1)
    zeros_nd = jnp.zeros((N, D), jnp.float32)

    hv = []
    for k in range(K):
        sl = slice(k * EH, (k + 1) * EH)
        hv.append(_sc_gather(hidden, ent_emb, _pad3(idx_vi[sl]),
                             _pad3(idx_vj[sl]), idx_ent3, e_per_w, ent_w,
                             do_ent=(k == 0)))
    ent_g = hv[0][2]
    parts = []
    for k in range(K):
        sl = slice(k * EH, (k + 1) * EH)
        am = _tc_edge(hv[k][0], hv[k][1], rel2d[sl], tatt2d[sl], rel_emb,
                      f_msg_ws, f_msg_b, g_msg_W, g_msg_b, f_att_ws)
        parts.append(_sc_scatter(am, _pad3(idx_vj[sl]), zeros_nd, e_per_w))
    return _tc_node(parts[0], parts[1], hidden, ent_g, hidden_uncon, natt2d,
                    proj_W, proj_b, f_hid_ws, f_hid_b, g_hid_W, g_hid_b,
                    gru_W, gru_b)


# speedup vs baseline: 1.4241x; 1.4241x over previous
import functools

import jax
import jax.numpy as jnp
from jax import lax
from jax.experimental import pallas as pl
from jax.experimental.pallas import tpu as pltpu
from jax.experimental.pallas import tpu_sc as plsc


def _sc_gather(hidden, ent_emb, idx_vi3, idx_vj3, idx_ent3, e_per_w, ent_w):
    NW, n_ch, CH = idx_vi3.shape
    E = NW * e_per_w
    N, D = hidden.shape
    NE = ent_emb.shape[0]
    n_full = e_per_w // CH
    tail = e_per_w - n_full * CH
    assert n_ch == n_full + (1 if tail else 0)
    ENT_W, ent_ch, _ = idx_ent3.shape

    mesh = plsc.VectorSubcoreMesh(core_axis_name="c", subcore_axis_name="s")

    @functools.partial(
        pl.kernel,
        out_type=(
            jax.ShapeDtypeStruct((E, D), jnp.float32),
            jax.ShapeDtypeStruct((E, D), jnp.float32),
            jax.ShapeDtypeStruct((NE, D), jnp.float32),
        ),
        mesh=mesh,
        scratch_types=[
            pltpu.VMEM((n_ch, CH), jnp.int32),
            pltpu.VMEM((n_ch, CH), jnp.int32),
            pltpu.VMEM((3, CH, D), jnp.float32),
            pltpu.VMEM((3, CH, D), jnp.float32),
            pltpu.SemaphoreType.DMA((3,)),
            pltpu.SemaphoreType.DMA((3,)),
        ],
    )
    def gather_kernel(hid_h, ent_h, ivi_h, ivj_h, ient_h,
                      ovi_h, ovj_h, oent_h,
                      ivi_v, ivj_v, rvi_v, rvj_v, g_sem, w_sem):
        wid = lax.axis_index("s") * 2 + lax.axis_index("c")
        base = wid * e_per_w

        pltpu.sync_copy(ivi_h.at[wid], ivi_v)
        pltpu.sync_copy(ivj_h.at[wid], ivj_v)

        def start_gather(i, slot):
            pltpu.async_copy(hid_h.at[ivi_v.at[i]], rvi_v.at[slot],
                             g_sem.at[slot])
            pltpu.async_copy(hid_h.at[ivj_v.at[i]], rvj_v.at[slot],
                             g_sem.at[slot])

        def wait_gather(i, slot):
            pltpu.make_async_copy(hid_h.at[ivi_v.at[i]], rvi_v.at[slot],
                                  g_sem.at[slot]).wait()
            pltpu.make_async_copy(hid_h.at[ivj_v.at[i]], rvj_v.at[slot],
                                  g_sem.at[slot]).wait()

        def start_write(i, slot, w=CH):
            off = base + i * CH
            pltpu.async_copy(rvi_v.at[slot, pl.ds(0, w)],
                             ovi_h.at[pl.ds(off, w)], w_sem.at[slot])
            pltpu.async_copy(rvj_v.at[slot, pl.ds(0, w)],
                             ovj_h.at[pl.ds(off, w)], w_sem.at[slot])

        def wait_write(i, slot, w=CH):
            off = base + i * CH
            pltpu.make_async_copy(rvi_v.at[slot, pl.ds(0, w)],
                                  ovi_h.at[pl.ds(off, w)],
                                  w_sem.at[slot]).wait()
            pltpu.make_async_copy(rvj_v.at[slot, pl.ds(0, w)],
                                  ovj_h.at[pl.ds(off, w)],
                                  w_sem.at[slot]).wait()

        start_gather(0, 0)
        start_gather(1, 1)

        def ebody(i, _):
            slot = lax.rem(i, 3)
            wait_gather(i, slot)
            start_write(i, slot)

            @pl.when(i + 2 < n_ch)
            def _():
                fslot = lax.rem(i + 2, 3)

                @pl.when(i >= 1)
                def _():
                    wait_write(i - 1, fslot)
                start_gather(i + 2, fslot)
            return ()

        lax.fori_loop(0, n_full, ebody, (), unroll=False)
        if tail:
            tslot = n_full % 3
            wait_gather(n_full, tslot)
            start_write(n_full, tslot, tail)
            wait_write(n_full, tslot, tail)
        else:
            wait_write(n_full - 3, (n_full - 3) % 3)
        wait_write(n_full - 2, (n_full - 2) % 3)
        wait_write(n_full - 1, (n_full - 1) % 3)

        @pl.when(wid < ENT_W)
        def _():
            pltpu.sync_copy(ient_h.at[wid], ivi_v.at[pl.ds(0, ent_ch)])

            ech = ent_w // ent_ch

            def nbody(i, _):
                off = wid * ent_w + i * ech
                pltpu.async_copy(ent_h.at[ivi_v.at[i]], rvi_v.at[0],
                                 g_sem.at[0]).wait()
                pltpu.sync_copy(rvi_v.at[0, pl.ds(0, ech)],
                                oent_h.at[pl.ds(off, ech)])
                return ()
            lax.fori_loop(0, ent_ch, nbody, (), unroll=False)

    return gather_kernel(hidden, ent_emb, idx_vi3, idx_vj3, idx_ent3)


def _tc_edge(hvi, hvj, rel2d, tatt2d, rel_emb, f_msg_ws, f_msg_b,
             g_msg_W, g_msg_b, f_att_ws):
    E, D = hvi.shape
    R = rel_emb.shape[0]
    BLK = 8000
    n_blk = E // BLK
    assert n_blk * BLK == E

    def body(hvi_r, hvj_r, rel_r, tatt_r, remb_r, fw_r, fb_r, gw_r, gb_r,
             aw_r, out_r):
        rel = rel_r[...]
        onehot = (rel == lax.broadcasted_iota(jnp.int32, (BLK, R), 1)
                  ).astype(jnp.float32)
        rel_v = jnp.dot(onehot, remb_r[...],
                        preferred_element_type=jnp.float32)
        hvi_b = hvi_r[...]
        hvj_b = hvj_r[...]
        w0 = fw_r[0]
        w1 = fw_r[1]
        w2 = fw_r[2]
        msg = jnp.tanh(hvj_b * (w0 + rel_v * (w1 + hvi_b * w2)) + fb_r[...])
        msg = msg + jnp.tanh(
            jnp.dot(msg, gw_r[...], preferred_element_type=jnp.float32)
            + gb_r[...])
        out_r[...] = msg * tatt_r[...] * aw_r[...]

    return pl.pallas_call(
        body,
        grid=(n_blk,),
        in_specs=[
            pl.BlockSpec((BLK, D), lambda i: (i, 0)),
            pl.BlockSpec((BLK, D), lambda i: (i, 0)),
            pl.BlockSpec((BLK, 1), lambda i: (i, 0)),
            pl.BlockSpec((BLK, 1), lambda i: (i, 0)),
            pl.BlockSpec((R, D), lambda i: (0, 0)),
            pl.BlockSpec((3, D), lambda i: (0, 0)),
            pl.BlockSpec((D,), lambda i: (0,)),
            pl.BlockSpec((D, D), lambda i: (0, 0)),
            pl.BlockSpec((D,), lambda i: (0,)),
            pl.BlockSpec((1, D), lambda i: (0, 0)),
        ],
        out_specs=pl.BlockSpec((BLK, D), lambda i: (i, 0)),
        out_shape=jax.ShapeDtypeStruct((E, D), jnp.float32),
    )(hvi, hvj, rel2d, tatt2d, rel_emb, f_msg_ws, f_msg_b, g_msg_W,
      g_msg_b, f_att_ws)


def _sc_scatter(att_msg, idx_dst3, zeros_nd, e_per_w):
    E, D = att_msg.shape
    N = zeros_nd.shape[0]
    NW, n_ch, CH = idx_dst3.shape
    NS = 16
    n_full = e_per_w // CH
    tail = e_per_w - n_full * CH
    assert n_ch == n_full + (1 if tail else 0) and e_per_w * NW == E
    rows_per_s = (N // NS) // 8 * 8
    rows_tail = N - NS * rows_per_s

    mesh = plsc.VectorSubcoreMesh(core_axis_name="c", subcore_axis_name="s")

    @functools.partial(
        pl.kernel,
        out_type=jax.ShapeDtypeStruct((2 * N, D), jnp.float32),
        mesh=mesh,
        scratch_types=[
            pltpu.VMEM_SHARED((N, D), jnp.float32),
            pltpu.VMEM((n_ch, CH), jnp.int32),
            pltpu.VMEM((2, CH, D), jnp.float32),
            pltpu.SemaphoreType.DMA((2,)),
        ],
    )
    def scatter_kernel(msg_h, idx_h, zero_h, out_h, acc_sh, idx_v, rows_v,
                       l_sem):
        cid = lax.axis_index("c")
        sid = lax.axis_index("s")
        wid = sid * 2 + cid
        base = wid * e_per_w

        pltpu.sync_copy(idx_h.at[wid], idx_v)

        def start_load(i, slot, w=CH):
            off = base + i * CH
            pltpu.async_copy(msg_h.at[pl.ds(off, w)],
                             rows_v.at[slot, pl.ds(0, w)], l_sem.at[slot])

        def wait_load(i, slot, w=CH):
            off = base + i * CH
            pltpu.make_async_copy(msg_h.at[pl.ds(off, w)],
                                  rows_v.at[slot, pl.ds(0, w)],
                                  l_sem.at[slot]).wait()

        @pl.when(sid == 0)
        def _():
            pltpu.sync_copy(zero_h, acc_sh)
        start_load(0, 0)
        plsc.subcore_barrier()

        def ebody(i, _):
            slot = lax.rem(i, 2)

            @pl.when(i + 1 < n_full)
            def _():
                start_load(i + 1, lax.rem(i + 1, 2))
            wait_load(i, slot)
            pltpu.sync_copy(rows_v.at[slot], acc_sh.at[idx_v.at[i]],
                            add=True)
            return ()

        lax.fori_loop(0, n_full, ebody, (), unroll=False)
        if tail:
            tslot = n_full % 2
            start_load(n_full, tslot, tail)
            wait_load(n_full, tslot, tail)
            pltpu.sync_copy(rows_v.at[tslot, pl.ds(0, tail)],
                            acc_sh.at[idx_v.at[n_full, pl.ds(0, tail)]],
                            add=True)
        plsc.subcore_barrier()

        out_base = cid * N + sid * rows_per_s
        pltpu.sync_copy(acc_sh.at[pl.ds(sid * rows_per_s, rows_per_s)],
                        out_h.at[pl.ds(out_base, rows_per_s)])

        @pl.when(sid == 0)
        def _():
            pltpu.sync_copy(
                acc_sh.at[pl.ds(NS * rows_per_s, rows_tail)],
                out_h.at[pl.ds(cid * N + NS * rows_per_s, rows_tail)])

    return scatter_kernel(att_msg, idx_dst3, zeros_nd)


def _tc_node(partials, hidden, ent_g, hidden_uncon, natt2d, proj_W, proj_b,
             f_hid_ws, f_hid_b, g_hid_W, g_hid_b, gru_W, gru_b):
    N, D = hidden.shape
    BLK = 1000
    n_blk = N // BLK
    assert n_blk * BLK == N

    def body(p0_r, p1_r, hid_r, ent_r, hu_r, natt_r, pw_r, pb_r, fw_r, fb_r,
             gw_r, gb_r, uw_r, ub_r, out_r):
        u = p0_r[...] + p1_r[...]
        h = hid_r[...]
        ent = ent_r[...]
        uc = jnp.tanh(
            jnp.dot(hu_r[...], pw_r[...], preferred_element_type=jnp.float32)
            + pb_r[...]) * natt_r[...]
        cand = jnp.tanh(
            u * (fw_r[0] + ent * fw_r[1] + uc * fw_r[2])
            + h * (fw_r[3] + ent * fw_r[4] + uc * fw_r[5])
            + ent * fw_r[6] + uc * fw_r[7] + ent * uc * fw_r[8]
            + fb_r[...])
        cand = cand + jnp.tanh(
            jnp.dot(cand, gw_r[...], preferred_element_type=jnp.float32)
            + gb_r[...])
        z = jax.nn.sigmoid(
            jnp.dot(h, uw_r[0], preferred_element_type=jnp.float32)
            + jnp.dot(cand, uw_r[1], preferred_element_type=jnp.float32)
            + ub_r[...])
        out_r[...] = (1.0 - z) * h + z * cand

    gru_W3 = gru_W.reshape(2, D, D)

    return pl.pallas_call(
        body,
        grid=(n_blk,),
        in_specs=[
            pl.BlockSpec((BLK, D), lambda i: (i, 0)),
            pl.BlockSpec((BLK, D), lambda i: (i + n_blk, 0)),
            pl.BlockSpec((BLK, D), lambda i: (i, 0)),
            pl.BlockSpec((BLK, D), lambda i: (i, 0)),
            pl.BlockSpec((BLK, D), lambda i: (i, 0)),
            pl.BlockSpec((BLK, 1), lambda i: (i, 0)),
            pl.BlockSpec((D, D), lambda i: (0, 0)),
            pl.BlockSpec((D,), lambda i: (0,)),
            pl.BlockSpec((9, D), lambda i: (0, 0)),
            pl.BlockSpec((D,), lambda i: (0,)),
            pl.BlockSpec((D, D), lambda i: (0, 0)),
            pl.BlockSpec((D,), lambda i: (0,)),
            pl.BlockSpec((2, D, D), lambda i: (0, 0, 0)),
            pl.BlockSpec((D,), lambda i: (0,)),
        ],
        out_specs=pl.BlockSpec((BLK, D), lambda i: (i, 0)),
        out_shape=jax.ShapeDtypeStruct((N, D), jnp.float32),
    )(partials, partials, hidden, ent_g, hidden_uncon, natt2d, proj_W,
      proj_b, f_hid_ws, f_hid_b, g_hid_W, g_hid_b, gru_W3, gru_b)


def kernel(hidden, seen_edges, trans_attention, node_attention, hidden_uncon,
           memorized_nodes, ent_emb, rel_emb, f_msg_ws, f_msg_b, g_msg_W,
           g_msg_b, f_att_ws, f_hid_ws, f_hid_b, g_hid_W, g_hid_b, proj_W,
           proj_b, gru_W, gru_b):
    E = seen_edges.shape[0]
    N, D = hidden.shape

    NW, CH = 32, 80
    e_per_w = E // NW
    n_ch = -(-e_per_w // CH)

    def _pad3(idx):
        a = idx.reshape(NW, e_per_w)
        a = jnp.pad(a, ((0, 0), (0, n_ch * CH - e_per_w)))
        return a.reshape(NW, n_ch, CH)

    idx_vi3 = _pad3(seen_edges[:, 6].astype(jnp.int32))
    idx_vj3 = _pad3(seen_edges[:, 7].astype(jnp.int32))
    rel2d = seen_edges[:, 3].astype(jnp.int32).reshape(E, 1)
    ENT_W, ent_w = 25, N // 25
    idx_ent3 = memorized_nodes[:, 1].astype(jnp.int32).reshape(ENT_W, -1, CH)
    tatt2d = trans_attention.reshape(E, 1)
    natt2d = node_attention.reshape(N, 1)

    hvi, hvj, ent_g = _sc_gather(hidden, ent_emb, idx_vi3, idx_vj3, idx_ent3,
                                 e_per_w, ent_w)
    att_msg = _tc_edge(hvi, hvj, rel2d, tatt2d, rel_emb, f_msg_ws, f_msg_b,
                       g_msg_W, g_msg_b, f_att_ws)
    partials = _sc_scatter(att_msg, idx_vj3, jnp.zeros((N, D), jnp.float32),
                           e_per_w)
    return _tc_node(partials, hidden, ent_g, hidden_uncon, natt2d, proj_W,
                    proj_b, f_hid_ws, f_hid_b, g_hid_W, g_hid_b, gru_W, gru_b)
